# baseline (speedup 1.0000x reference)
import jax
import jax.numpy as jnp
from jax.experimental import pallas as pl

TM = 1024


def _pack(t, swap):
    tsq = jnp.sum(t * t, axis=0, keepdims=True)
    one = jnp.ones_like(tsq)
    if swap:
        c = jnp.concatenate([t, one, tsq], axis=0)
    else:
        c = jnp.concatenate([t * -2.0, tsq, one], axis=0)
    ch = c.astype(jnp.bfloat16).astype(jnp.float32)
    parts = [ch, ch, c - ch] if not swap else [ch, c - ch, ch]
    return jnp.concatenate(parts, axis=0).astype(jnp.bfloat16)


def _chamfer_kernel(xt_ref, yt_ref, rowmin_ref, colmin_ref):
    N = xt_ref.shape[2]
    M = yt_ref.shape[2]
    a2t = _pack(xt_ref[0], swap=False)
    b2 = _pack(yt_ref[0], swap=True)

    rm128 = None
    for j in range(M // TM):
        d = jax.lax.dot_general(
            a2t, b2[:, j * TM:(j + 1) * TM],
            (((0,), (0,)), ((), ())),
            preferred_element_type=jnp.float32)
        colmin_ref[0, 0, j * TM:(j + 1) * TM] = jnp.min(d, axis=0)
        p = d[:, 0:128]
        for k in range(1, TM // 128):
            p = jnp.minimum(p, d[:, k * 128:(k + 1) * 128])
        rm128 = p if rm128 is None else jnp.minimum(rm128, p)
    rmT = jnp.swapaxes(rm128, 0, 1)
    q = rmT[0:8]
    for k in range(1, 16):
        q = jnp.minimum(q, rmT[8 * k:8 * (k + 1)])
    rowmin_ref[0, 0] = jnp.min(q, axis=0)


def kernel(x, y):
    B, N, _ = x.shape
    M = y.shape[1]
    xt = jnp.transpose(x, (0, 2, 1))
    yt = jnp.transpose(y, (0, 2, 1))
    rowmin, colmin = pl.pallas_call(
        _chamfer_kernel,
        grid=(B,),
        in_specs=[
            pl.BlockSpec((1, 3, N), lambda b: (b, 0, 0)),
            pl.BlockSpec((1, 3, M), lambda b: (b, 0, 0)),
        ],
        out_specs=[
            pl.BlockSpec((1, 1, N), lambda b: (b, 0, 0)),
            pl.BlockSpec((1, 1, M), lambda b: (b, 0, 0)),
        ],
        out_shape=[
            jax.ShapeDtypeStruct((B, 1, N), jnp.float32),
            jax.ShapeDtypeStruct((B, 1, M), jnp.float32),
        ],
    )(xt, yt)
    return jnp.mean(rowmin) + jnp.mean(colmin)
